# baseline (speedup 1.0000x reference)
import functools

import jax
import jax.numpy as jnp
import numpy as np
from jax import lax
from jax.experimental import pallas as pl
from jax.experimental.pallas import tpu as pltpu
from jax.experimental.pallas import tpu_sc as plsc

FEAT = 128
HEADS = 2
N_RBF = 20
CUTOFF = 5.0
RBF_PAD = 32

_NC = 2
_NS = 16
_NW = _NC * _NS
_CHUNK = 128


def _swish(x):
    return x * jax.nn.sigmoid(x)


def _proj_body(x_ref, wq_ref, bq_ref, wkv_ref, bkv_ref, q_ref, kv_ref):
    x = x_ref[...]
    q_ref[...] = jnp.dot(x, wq_ref[...], preferred_element_type=jnp.float32) + bq_ref[...]
    kv_ref[...] = jnp.dot(x, wkv_ref[...], preferred_element_type=jnp.float32) + bkv_ref[...]


def _node_proj(x_i, wq_p, bq_p, wkv_p, bkv_p):
    n = x_i.shape[0]
    blk = 1000
    grid = n // blk
    return pl.pallas_call(
        _proj_body,
        grid=(grid,),
        in_specs=[
            pl.BlockSpec((blk, FEAT), lambda i: (i, 0)),
            pl.BlockSpec((FEAT, HEADS * FEAT), lambda i: (0, 0)),
            pl.BlockSpec((1, HEADS * FEAT), lambda i: (0, 0)),
            pl.BlockSpec((FEAT, 2 * HEADS * FEAT), lambda i: (0, 0)),
            pl.BlockSpec((1, 2 * HEADS * FEAT), lambda i: (0, 0)),
        ],
        out_specs=[
            pl.BlockSpec((blk, HEADS * FEAT), lambda i: (i, 0)),
            pl.BlockSpec((blk, 2 * HEADS * FEAT), lambda i: (i, 0)),
        ],
        out_shape=[
            jax.ShapeDtypeStruct((n, HEADS * FEAT), jnp.float32),
            jax.ShapeDtypeStruct((n, 2 * HEADS * FEAT), jnp.float32),
        ],
    )(x_i, wq_p, bq_p, wkv_p, bkv_p)


def _make_gather(e_pad, dq, dkv):
    epw = e_pad // _NW
    nch = epw // _CHUNK
    mesh = plsc.VectorSubcoreMesh(
        core_axis_name="c", subcore_axis_name="s",
        num_cores=_NC, num_subcores=_NS)

    @functools.partial(
        pl.kernel,
        mesh=mesh,
        out_type=[
            jax.ShapeDtypeStruct((e_pad, dq), jnp.float32),
            jax.ShapeDtypeStruct((e_pad, dkv), jnp.float32),
        ],
        scratch_types=[
            pltpu.VMEM((_CHUNK,), jnp.int32),
            pltpu.VMEM((_CHUNK,), jnp.int32),
            pltpu.VMEM((_CHUNK, dq), jnp.float32),
            pltpu.VMEM((_CHUNK, dkv), jnp.float32),
            pltpu.SemaphoreType.DMA,
            pltpu.SemaphoreType.DMA,
        ],
    )
    def gather_k(qtab, kvtab, idx_i, idx_j, qout, kvout,
                 idxi_v, idxj_v, qbuf, kvbuf, sq, skv):
        wid = lax.axis_index("s") * _NC + lax.axis_index("c")
        base = wid * epw

        def body(c, carry):
            off = pl.multiple_of(base + c * _CHUNK, _CHUNK)
            pltpu.sync_copy(idx_i.at[pl.ds(off, _CHUNK)], idxi_v)
            pltpu.sync_copy(idx_j.at[pl.ds(off, _CHUNK)], idxj_v)
            cq = pltpu.async_copy(qtab.at[idxi_v], qbuf, sq)
            ckv = pltpu.async_copy(kvtab.at[idxj_v], kvbuf, skv)
            cq.wait()
            pltpu.sync_copy(qbuf, qout.at[pl.ds(off, _CHUNK)])
            ckv.wait()
            pltpu.sync_copy(kvbuf, kvout.at[pl.ds(off, _CHUNK)])
            return carry

        lax.fori_loop(0, nch, body, 0)

    return gather_k


def _edge_body(step, inv2s2, d_ref, qg_ref, kvg_ref, wd_ref, bd_ref, out_ref):
    mu = lax.broadcasted_iota(jnp.int32, (1, RBF_PAD), 1).astype(jnp.float32) * step
    d = d_ref[...]
    env = jnp.where(d <= CUTOFF,
                    0.5 * (jnp.cos(np.pi * d / CUTOFF) + 1.0), 0.0)
    ef = jnp.exp(-((d - mu) ** 2) * inv2s2) * env
    dd = jnp.dot(ef, wd_ref[...], preferred_element_type=jnp.float32) + bd_ref[...]
    dd = _swish(dd)
    q = qg_ref[...]
    kv = kvg_ref[...]
    prod = q * kv[:, :HEADS * FEAT] * dd[:, :HEADS * FEAT]
    w0 = _swish(jnp.sum(prod[:, :FEAT], axis=1, keepdims=True))
    w1 = _swish(jnp.sum(prod[:, FEAT:], axis=1, keepdims=True))
    vout = kv[:, HEADS * FEAT:] * dd[:, HEADS * FEAT:]
    out_ref[:, :FEAT] = vout[:, :FEAT] * w0
    out_ref[:, FEAT:] = vout[:, FEAT:] * w1


def _edge_stage(dist2d, qg, kvg, wd_p, bd_p):
    e_pad = dist2d.shape[0]
    be = 1024
    grid = e_pad // be
    sigma = CUTOFF / (N_RBF - 1)
    body = functools.partial(_edge_body, float(sigma),
                             float(1.0 / (2.0 * sigma * sigma)))
    return pl.pallas_call(
        body,
        grid=(grid,),
        in_specs=[
            pl.BlockSpec((be, 1), lambda i: (i, 0)),
            pl.BlockSpec((be, HEADS * FEAT), lambda i: (i, 0)),
            pl.BlockSpec((be, 2 * HEADS * FEAT), lambda i: (i, 0)),
            pl.BlockSpec((RBF_PAD, 2 * HEADS * FEAT), lambda i: (0, 0)),
            pl.BlockSpec((1, 2 * HEADS * FEAT), lambda i: (0, 0)),
        ],
        out_specs=pl.BlockSpec((be, HEADS * FEAT), lambda i: (i, 0)),
        out_shape=jax.ShapeDtypeStruct((e_pad, HEADS * FEAT), jnp.float32),
    )(dist2d, qg, kvg, wd_p, bd_p)


def _pack_node_w(w):
    return jnp.transpose(w, (2, 0, 1)).reshape(w.shape[2], w.shape[0] * w.shape[1])


def kernel(dist, nbrs, x_i, Wq, bq, Wk, bk, Wdk, bdk, Wv, bv, Wdv, bdv):
    e = dist.shape[0]
    e_pad = ((e + _NW * _CHUNK - 1) // (_NW * _CHUNK)) * (_NW * _CHUNK)

    wq_p = _pack_node_w(Wq)
    wkv_p = jnp.concatenate([_pack_node_w(Wk), _pack_node_w(Wv)], axis=1)
    bq_p = bq.reshape(1, HEADS * FEAT)
    bkv_p = jnp.concatenate([bk.reshape(1, -1), bv.reshape(1, -1)], axis=1)

    wdk_p = _pack_node_w(Wdk)
    wdv_p = _pack_node_w(Wdv)
    wd_p = jnp.zeros((RBF_PAD, 2 * HEADS * FEAT), jnp.float32)
    wd_p = wd_p.at[:N_RBF].set(jnp.concatenate([wdk_p, wdv_p], axis=1))
    bd_p = jnp.concatenate([bdk.reshape(1, -1), bdv.reshape(1, -1)], axis=1)

    qtab, kvtab = _node_proj(x_i, wq_p, bq_p, wkv_p, bkv_p)

    idx = nbrs.astype(jnp.int32)
    pad = e_pad - e
    idx_i = jnp.pad(idx[:, 0], (0, pad))
    idx_j = jnp.pad(idx[:, 1], (0, pad))
    qg, kvg = _make_gather(e_pad, HEADS * FEAT, 2 * HEADS * FEAT)(
        qtab, kvtab, idx_i, idx_j)

    dist2d = jnp.pad(dist, (0, pad), constant_values=1.0).reshape(e_pad, 1)
    out = _edge_stage(dist2d, qg, kvg, wd_p, bd_p)
    return out[:e]
